# baseline (speedup 1.0000x reference)
import functools

import jax
import jax.numpy as jnp
from jax import lax
from jax.experimental import pallas as pl
from jax.experimental.pallas import tpu as pltpu
from jax.experimental.pallas import tpu_sc as plsc

N = 10000
F = 128
H = 128
HH = H // 2
E = 320000

CH = 128
NCH = 160
NCHD = NCH // 2
EPW = NCH * CH
EPAD = 16 * EPW
NACC = 10240
ZR = NACC // 16
WR = 632
NOUT = 16 * WR
G = 2
NG = NCH // G
EPS = 1e-5
BLK = 1000
NB = N // BLK

_mesh = plsc.VectorSubcoreMesh(core_axis_name="c", subcore_axis_name="s")


def _deg_body(dst_hbm, out_hbm, dstv, ones_v, zero_v, dacc, dsem):
    c = lax.axis_index("c")
    s = lax.axis_index("s")
    pltpu.sync_copy(dst_hbm.at[s, pl.ds(c * NCHD, NCHD)], dstv)

    one16 = jnp.ones((16,), jnp.float32)
    zro16 = jnp.zeros((16,), jnp.float32)

    @pl.loop(0, 128)
    def _fill(r):
        ones_v[r] = one16
        zero_v[r] = zro16

    @pl.loop(0, ZR // 128)
    def _zero(k):
        pltpu.sync_copy(zero_v, dacc.at[pl.ds(s * ZR + k * 128, 128)])

    plsc.subcore_barrier()

    @pl.loop(0, NCHD)
    def _hist(j):
        pltpu.async_copy(ones_v, dacc.at[dstv.at[j]], dsem, add=True)

    @pl.loop(0, NCHD)
    def _drain(j):
        pltpu.make_async_copy(ones_v, dacc.at[dstv.at[j]], dsem).wait()

    plsc.subcore_barrier()
    pltpu.sync_copy(dacc.at[pl.ds(s * WR, WR)], out_hbm.at[c, pl.ds(s * WR, WR)])


_deg_call = pl.kernel(
    _deg_body,
    out_type=jax.ShapeDtypeStruct((2, NOUT, 16), jnp.float32),
    mesh=_mesh,
    scratch_types=[
        pltpu.VMEM((NCHD, CH), jnp.int32),
        pltpu.VMEM((CH, 16), jnp.float32),
        pltpu.VMEM((CH, 16), jnp.float32),
        pltpu.VMEM_SHARED((NACC, 16), jnp.float32),
        pltpu.SemaphoreType.DMA,
    ],
)


def _agg_body(tab_hbm, src_hbm, dst_hbm, out_hbm, srcv, dstv, rows, zero_v, acc,
              gsemA, gsemB, ssemA, ssemB):
    c = lax.axis_index("c")
    s = lax.axis_index("s")
    pltpu.sync_copy(src_hbm.at[s], srcv)
    pltpu.sync_copy(dst_hbm.at[s], dstv)

    zro16 = jnp.zeros((16,), jnp.float32)

    @pl.loop(0, 64)
    def _fill(r):
        for q in range(HH // 16):
            zero_v[r, pl.ds(q * 16, 16)] = zro16

    @pl.loop(0, ZR // 64)
    def _zero(k):
        pltpu.sync_copy(zero_v, acc.at[pl.ds(s * ZR + k * 64, 64)])

    plsc.subcore_barrier()
    tab = tab_hbm.at[c]

    def fire_g(g, h, sem):
        for b in range(G):
            pltpu.async_copy(tab.at[srcv.at[g * G + b]], rows.at[h, b], sem)

    def wait_g(g, h, sem):
        for b in range(G):
            pltpu.make_async_copy(tab.at[srcv.at[g * G + b]], rows.at[h, b], sem).wait()

    def fire_s(g, h, sem):
        for b in range(G):
            pltpu.async_copy(rows.at[h, b], acc.at[dstv.at[g * G + b]], sem, add=True)

    def wait_s(g, h, sem):
        for b in range(G):
            pltpu.make_async_copy(rows.at[h, b], acc.at[dstv.at[g * G + b]], sem).wait()

    fire_g(0, 0, gsemA)
    fire_g(1, 1, gsemB)

    @pl.loop(0, NG // 2 - 1)
    def _edges(k):
        g0 = 2 * k
        g1 = 2 * k + 1
        wait_g(g0, 0, gsemA)
        fire_s(g0, 0, ssemA)
        wait_g(g1, 1, gsemB)
        fire_s(g1, 1, ssemB)
        wait_s(g0, 0, ssemA)
        fire_g(g0 + 2, 0, gsemA)
        wait_s(g1, 1, ssemB)
        fire_g(g1 + 2, 1, gsemB)

    gl = NG - 2
    wait_g(gl, 0, gsemA)
    fire_s(gl, 0, ssemA)
    wait_g(gl + 1, 1, gsemB)
    fire_s(gl + 1, 1, ssemB)
    wait_s(gl, 0, ssemA)
    wait_s(gl + 1, 1, ssemB)

    plsc.subcore_barrier()
    pltpu.sync_copy(acc.at[pl.ds(s * WR, WR)],
                    out_hbm.at[pl.ds(s * WR, WR), pl.ds(c * HH, HH)])


_agg_call = pl.kernel(
    _agg_body,
    out_type=jax.ShapeDtypeStruct((NOUT, H), jnp.float32),
    mesh=_mesh,
    scratch_types=[
        pltpu.VMEM((NCH, CH), jnp.int32),
        pltpu.VMEM((NCH, CH), jnp.int32),
        pltpu.VMEM((2, G, CH, HH), jnp.float32),
        pltpu.VMEM((64, HH), jnp.float32),
        pltpu.VMEM_SHARED((NACC, HH), jnp.float32),
        pltpu.SemaphoreType.DMA,
        pltpu.SemaphoreType.DMA,
        pltpu.SemaphoreType.DMA,
        pltpu.SemaphoreType.DMA,
    ],
    compiler_params=pltpu.CompilerParams(use_tc_tiling_on_sc=False),
)


def _prep_kernel(degp_ref, x_ref, w1_ref, hs1_ref, dinv_ref):
    deg = degp_ref[0] + degp_ref[1]
    dinv = lax.rsqrt(deg[:, 0:1] + 1.0)
    h = jnp.dot(x_ref[...], w1_ref[...], preferred_element_type=jnp.float32)
    hs = h * dinv
    hs1_ref[0] = hs[:, :HH]
    hs1_ref[1] = hs[:, HH:]
    dinv_ref[...] = jnp.broadcast_to(dinv, (BLK, 16))


def _prep_call(degp, x, w1):
    return pl.pallas_call(
        _prep_kernel,
        grid=(NB,),
        in_specs=[
            pl.BlockSpec((2, BLK, 16), lambda j: (0, j, 0)),
            pl.BlockSpec((BLK, F), lambda j: (j, 0)),
            pl.BlockSpec((F, H), lambda j: (0, 0)),
        ],
        out_specs=[
            pl.BlockSpec((2, BLK, HH), lambda j: (0, j, 0)),
            pl.BlockSpec((BLK, 16), lambda j: (j, 0)),
        ],
        out_shape=[
            jax.ShapeDtypeStruct((2, N, HH), jnp.float32),
            jax.ShapeDtypeStruct((N, 16), jnp.float32),
        ],
    )(degp, x, w1)


def _bn_kernel(aggp_ref, hs_ref, dinv_ref, b_ref, g_ref, beta_ref, w2_ref,
               out_ref, z_s, stat_s, *, matmul):
    p = pl.program_id(0)
    j = pl.program_id(1)

    @pl.when(p == 0)
    def _stats():
        dinv = dinv_ref[...][:, 0:1]
        hs = jnp.concatenate([hs_ref[0], hs_ref[1]], axis=1)
        z = dinv * (aggp_ref[...] + hs) + b_ref[...]
        z_s[pl.ds(j * BLK, BLK), :] = z

        @pl.when(j == 0)
        def _init():
            stat_s[...] = jnp.zeros_like(stat_s)

        stat_s[0:1, :] += jnp.sum(z, axis=0, keepdims=True)
        stat_s[1:2, :] += jnp.sum(z * z, axis=0, keepdims=True)

        @pl.when(j == NB - 1)
        def _finalize():
            mu = stat_s[0:1, :] * (1.0 / N)
            var = stat_s[1:2, :] * (1.0 / N) - mu * mu
            a = g_ref[...] * lax.rsqrt(var + EPS)
            stat_s[0:1, :] = a
            stat_s[1:2, :] = beta_ref[...] - a * mu

    @pl.when(p == 1)
    def _apply():
        a = stat_s[0:1, :]
        cshift = stat_s[1:2, :]
        y = a * z_s[pl.ds(j * BLK, BLK), :] + cshift
        if matmul:
            h2 = jnp.dot(y, w2_ref[...], preferred_element_type=jnp.float32)
            hs2 = h2 * dinv_ref[...][:, 0:1]
            out_ref[0] = hs2[:, :HH]
            out_ref[1] = hs2[:, HH:]
        else:
            out_ref[...] = y


def _bn_call(aggp, hs, dinv16, b, g, beta, w2, matmul):
    if matmul:
        out_spec = pl.BlockSpec((2, BLK, HH), lambda p, j: (0, j, 0))
        out_shape = jax.ShapeDtypeStruct((2, N, HH), jnp.float32)
    else:
        out_spec = pl.BlockSpec((BLK, H), lambda p, j: (j, 0))
        out_shape = jax.ShapeDtypeStruct((N, H), jnp.float32)
    return pl.pallas_call(
        functools.partial(_bn_kernel, matmul=matmul),
        grid=(2, NB),
        in_specs=[
            pl.BlockSpec((BLK, H), lambda p, j: (j, 0)),
            pl.BlockSpec((2, BLK, HH), lambda p, j: (0, j, 0)),
            pl.BlockSpec((BLK, 16), lambda p, j: (j, 0)),
            pl.BlockSpec((1, H), lambda p, j: (0, 0)),
            pl.BlockSpec((1, H), lambda p, j: (0, 0)),
            pl.BlockSpec((1, H), lambda p, j: (0, 0)),
            pl.BlockSpec((H, H), lambda p, j: (0, 0)),
        ],
        out_specs=out_spec,
        out_shape=out_shape,
        scratch_shapes=[
            pltpu.VMEM((N, H), jnp.float32),
            pltpu.VMEM((2, H), jnp.float32),
        ],
    )(aggp, hs, dinv16, b, g, beta, w2)


def kernel(x, edge_index, W1, b1, g1, beta1, W2, b2, g2, beta2):
    src = edge_index[0]
    dst = edge_index[1]
    npad = EPAD - E
    ar = jnp.arange(npad, dtype=jnp.int32)
    pad_src = ar & 8191
    pad_dst = N + (ar & 127)
    src_p = jnp.concatenate([src, pad_src]).reshape(16, NCH, CH)
    dst_p = jnp.concatenate([dst, pad_dst]).reshape(16, NCH, CH)

    degp = _deg_call(dst_p)
    hs1, dinv16 = _prep_call(degp, x, W1)
    agg1p = _agg_call(hs1, src_p, dst_p)
    hs2 = _bn_call(agg1p, hs1, dinv16, b1.reshape(1, H), g1.reshape(1, H),
                   beta1.reshape(1, H), W2, matmul=True)
    agg2p = _agg_call(hs2, src_p, dst_p)
    out = _bn_call(agg2p, hs2, dinv16, b2.reshape(1, H), g2.reshape(1, H),
                   beta2.reshape(1, H), W2, matmul=False)
    return out


# speedup vs baseline: 1.1414x; 1.1414x over previous
import functools

import jax
import jax.numpy as jnp
from jax import lax
from jax.experimental import pallas as pl
from jax.experimental.pallas import tpu as pltpu
from jax.experimental.pallas import tpu_sc as plsc

N = 10000
F = 128
H = 128
HH = H // 2
E = 320000

CH = 128
NCH = 160
NCHD = NCH // 2
EPW = NCH * CH
EPAD = 16 * EPW
NACC = 10240
ZR = NACC // 16
WR = 632
NOUT = 16 * WR
G = 2
NG = NCH // G
EPS = 1e-5
BLK = 1000
NB = N // BLK

_mesh = plsc.VectorSubcoreMesh(core_axis_name="c", subcore_axis_name="s")


def _deg_body(dst_hbm, out_hbm, dstv, ones_v, zero_v, dacc):
    c = lax.axis_index("c")
    s = lax.axis_index("s")
    pltpu.sync_copy(dst_hbm.at[s, pl.ds(c * NCHD, NCHD)], dstv)

    one16 = jnp.ones((16,), jnp.float32)
    zro16 = jnp.zeros((16,), jnp.float32)

    @pl.loop(0, 128)
    def _fill(r):
        ones_v[r] = one16
        zero_v[r] = zro16

    @pl.loop(0, ZR // 128)
    def _zero(k):
        pltpu.sync_copy(zero_v, dacc.at[pl.ds(s * ZR + k * 128, 128)])

    plsc.subcore_barrier()

    @pl.loop(0, NCHD)
    def _hist(j):
        pltpu.sync_copy(ones_v, dacc.at[dstv.at[j]], add=True)

    plsc.subcore_barrier()
    pltpu.sync_copy(dacc.at[pl.ds(s * WR, WR)], out_hbm.at[c, pl.ds(s * WR, WR)])


_deg_call = pl.kernel(
    _deg_body,
    out_type=jax.ShapeDtypeStruct((2, NOUT, 16), jnp.float32),
    mesh=_mesh,
    scratch_types=[
        pltpu.VMEM((NCHD, CH), jnp.int32),
        pltpu.VMEM((CH, 16), jnp.float32),
        pltpu.VMEM((CH, 16), jnp.float32),
        pltpu.VMEM_SHARED((NACC, 16), jnp.float32),
    ],
    compiler_params=pltpu.CompilerParams(use_tc_tiling_on_sc=False),
)


def _agg_body(tab_hbm, src_hbm, dst_hbm, out_hbm, srcv, dstv, rows, zero_v, acc,
              gsemA, gsemB):
    c = lax.axis_index("c")
    s = lax.axis_index("s")
    pltpu.sync_copy(src_hbm.at[s], srcv)
    pltpu.sync_copy(dst_hbm.at[s], dstv)

    zro16 = jnp.zeros((16,), jnp.float32)

    @pl.loop(0, 64)
    def _fill(r):
        for q in range(HH // 16):
            zero_v[r, pl.ds(q * 16, 16)] = zro16

    @pl.loop(0, ZR // 64)
    def _zero(k):
        pltpu.sync_copy(zero_v, acc.at[pl.ds(s * ZR + k * 64, 64)])

    plsc.subcore_barrier()
    tab = tab_hbm.at[c]

    def fire_g(g, h, sem):
        for b in range(G):
            pltpu.async_copy(tab.at[srcv.at[g * G + b]], rows.at[h, b], sem)

    def wait_g(g, h, sem):
        for b in range(G):
            pltpu.make_async_copy(tab.at[srcv.at[g * G + b]], rows.at[h, b], sem).wait()

    def scat(g, h):
        for b in range(G):
            pltpu.sync_copy(rows.at[h, b], acc.at[dstv.at[g * G + b]], add=True)

    fire_g(0, 0, gsemA)
    fire_g(1, 1, gsemB)

    @pl.loop(0, NG // 2 - 1)
    def _edges(k):
        g0 = 2 * k
        g1 = 2 * k + 1
        wait_g(g0, 0, gsemA)
        scat(g0, 0)
        fire_g(g0 + 2, 0, gsemA)
        wait_g(g1, 1, gsemB)
        scat(g1, 1)
        fire_g(g1 + 2, 1, gsemB)

    gl = NG - 2
    wait_g(gl, 0, gsemA)
    scat(gl, 0)
    wait_g(gl + 1, 1, gsemB)
    scat(gl + 1, 1)

    plsc.subcore_barrier()
    pltpu.sync_copy(acc.at[pl.ds(s * WR, WR)], out_hbm.at[c, pl.ds(s * WR, WR)])


_agg_call = pl.kernel(
    _agg_body,
    out_type=jax.ShapeDtypeStruct((2, NOUT, HH), jnp.float32),
    mesh=_mesh,
    scratch_types=[
        pltpu.VMEM((NCH, CH), jnp.int32),
        pltpu.VMEM((NCH, CH), jnp.int32),
        pltpu.VMEM((2, G, CH, HH), jnp.float32),
        pltpu.VMEM((64, HH), jnp.float32),
        pltpu.VMEM_SHARED((NACC, HH), jnp.float32),
        pltpu.SemaphoreType.DMA,
        pltpu.SemaphoreType.DMA,
    ],
    compiler_params=pltpu.CompilerParams(use_tc_tiling_on_sc=False),
)


def _prep_kernel(degp_ref, x_ref, w1_ref, hs1_ref, dinv_ref):
    deg = degp_ref[0] + degp_ref[1]
    dinv = lax.rsqrt(deg[:, 0:1] + 1.0)
    h = jnp.dot(x_ref[...], w1_ref[...], preferred_element_type=jnp.float32)
    hs = h * dinv
    hs1_ref[0] = hs[:, :HH]
    hs1_ref[1] = hs[:, HH:]
    dinv_ref[...] = jnp.broadcast_to(dinv, (BLK, 16))


def _prep_call(degp, x, w1):
    return pl.pallas_call(
        _prep_kernel,
        grid=(NB,),
        in_specs=[
            pl.BlockSpec((2, BLK, 16), lambda j: (0, j, 0)),
            pl.BlockSpec((BLK, F), lambda j: (j, 0)),
            pl.BlockSpec((F, H), lambda j: (0, 0)),
        ],
        out_specs=[
            pl.BlockSpec((2, BLK, HH), lambda j: (0, j, 0)),
            pl.BlockSpec((BLK, 16), lambda j: (j, 0)),
        ],
        out_shape=[
            jax.ShapeDtypeStruct((2, N, HH), jnp.float32),
            jax.ShapeDtypeStruct((N, 16), jnp.float32),
        ],
    )(degp, x, w1)


def _bn_kernel(aggp_ref, hs_ref, dinv_ref, b_ref, g_ref, beta_ref, w2_ref,
               out_ref, z_s, stat_s, *, matmul):
    p = pl.program_id(0)
    j = pl.program_id(1)

    @pl.when(p == 0)
    def _stats():
        dinv = dinv_ref[...][:, 0:1]
        hs = jnp.concatenate([hs_ref[0], hs_ref[1]], axis=1)
        agg = jnp.concatenate([aggp_ref[0], aggp_ref[1]], axis=1)
        z = dinv * (agg + hs) + b_ref[...]
        z_s[pl.ds(j * BLK, BLK), :] = z

        @pl.when(j == 0)
        def _init():
            stat_s[...] = jnp.zeros_like(stat_s)

        stat_s[0:1, :] += jnp.sum(z, axis=0, keepdims=True)
        stat_s[1:2, :] += jnp.sum(z * z, axis=0, keepdims=True)

        @pl.when(j == NB - 1)
        def _finalize():
            mu = stat_s[0:1, :] * (1.0 / N)
            var = stat_s[1:2, :] * (1.0 / N) - mu * mu
            a = g_ref[...] * lax.rsqrt(var + EPS)
            stat_s[0:1, :] = a
            stat_s[1:2, :] = beta_ref[...] - a * mu

    @pl.when(p == 1)
    def _apply():
        a = stat_s[0:1, :]
        cshift = stat_s[1:2, :]
        y = a * z_s[pl.ds(j * BLK, BLK), :] + cshift
        if matmul:
            h2 = jnp.dot(y, w2_ref[...], preferred_element_type=jnp.float32)
            hs2 = h2 * dinv_ref[...][:, 0:1]
            out_ref[0] = hs2[:, :HH]
            out_ref[1] = hs2[:, HH:]
        else:
            out_ref[...] = y


def _bn_call(aggp, hs, dinv16, b, g, beta, w2, matmul):
    if matmul:
        out_spec = pl.BlockSpec((2, BLK, HH), lambda p, j: (0, j, 0))
        out_shape = jax.ShapeDtypeStruct((2, N, HH), jnp.float32)
    else:
        out_spec = pl.BlockSpec((BLK, H), lambda p, j: (j, 0))
        out_shape = jax.ShapeDtypeStruct((N, H), jnp.float32)
    return pl.pallas_call(
        functools.partial(_bn_kernel, matmul=matmul),
        grid=(2, NB),
        in_specs=[
            pl.BlockSpec((2, BLK, HH), lambda p, j: (0, j, 0)),
            pl.BlockSpec((2, BLK, HH), lambda p, j: (0, j, 0)),
            pl.BlockSpec((BLK, 16), lambda p, j: (j, 0)),
            pl.BlockSpec((1, H), lambda p, j: (0, 0)),
            pl.BlockSpec((1, H), lambda p, j: (0, 0)),
            pl.BlockSpec((1, H), lambda p, j: (0, 0)),
            pl.BlockSpec((H, H), lambda p, j: (0, 0)),
        ],
        out_specs=out_spec,
        out_shape=out_shape,
        scratch_shapes=[
            pltpu.VMEM((N, H), jnp.float32),
            pltpu.VMEM((2, H), jnp.float32),
        ],
    )(aggp, hs, dinv16, b, g, beta, w2)


def kernel(x, edge_index, W1, b1, g1, beta1, W2, b2, g2, beta2):
    src = edge_index[0]
    dst = edge_index[1]
    npad = EPAD - E
    ar = jnp.arange(npad, dtype=jnp.int32)
    pad_src = (ar * 131) % N
    pad_dst = N + ar % (NACC - N)
    src_p = jnp.concatenate([src, pad_src]).reshape(16, NCH, CH)
    dst_p = jnp.concatenate([dst, pad_dst]).reshape(16, NCH, CH)

    degp = _deg_call(dst_p)
    hs1, dinv16 = _prep_call(degp, x, W1)
    agg1p = _agg_call(hs1, src_p, dst_p)
    hs2 = _bn_call(agg1p, hs1, dinv16, b1.reshape(1, H), g1.reshape(1, H),
                   beta1.reshape(1, H), W2, matmul=True)
    agg2p = _agg_call(hs2, src_p, dst_p)
    out = _bn_call(agg2p, hs2, dinv16, b2.reshape(1, H), g2.reshape(1, H),
                   beta2.reshape(1, H), W2, matmul=False)
    return out
